# Initial kernel scaffold; baseline (speedup 1.0000x reference)
import jax
import jax.numpy as jnp
from jax.experimental import pallas as pl


def kernel(x, gate_W, gate_b, W1, b1, W2, b2):
    raise NotImplementedError("write your pallas kernel here")


# speedup vs baseline: 1.1254x; 1.1254x over previous
import functools

import jax
import jax.numpy as jnp
from jax.experimental import pallas as pl
from jax.experimental.pallas import tpu as pltpu

N_TOKENS = 2048
D_MODEL = 1024
N_EXPERTS = 8
BT = 256


def _moe_dense_body(x_ref, gw_ref, gb_ref, W1_ref, b1_ref, W2_ref, b2_ref,
                    out_ref, g_scr):
    e = pl.program_id(1)
    x = x_ref[...]
    idx = jax.lax.broadcasted_iota(jnp.int32, (BT, N_EXPERTS), 1)

    @pl.when(e == 0)
    def _():
        logits = jax.lax.dot_general(
            x, gw_ref[...], (((1,), (1,)), ((), ())),
            preferred_element_type=jnp.float32) + gb_ref[...]
        m1 = jnp.max(logits, axis=1, keepdims=True)
        i1 = jnp.min(jnp.where(logits == m1, idx, N_EXPERTS), axis=1,
                     keepdims=True)
        masked = jnp.where(idx == i1, -jnp.inf, logits)
        m2 = jnp.max(masked, axis=1, keepdims=True)
        i2 = jnp.min(jnp.where(masked == m2, idx, N_EXPERTS), axis=1,
                     keepdims=True)
        t = jnp.exp(m2 - m1)
        p1 = 1.0 / (1.0 + t)
        p2 = t / (1.0 + t)
        g_scr[...] = jnp.where(idx == i1, p1,
                               jnp.where(idx == i2, p2, 0.0))
        out_ref[...] = jnp.zeros_like(out_ref)

    w = jnp.sum(g_scr[...] * (idx == e).astype(jnp.float32), axis=1,
                keepdims=True)
    h = jnp.maximum(
        jnp.dot(x, W1_ref[0], preferred_element_type=jnp.float32)
        + b1_ref[0], 0.0)
    o = (jnp.dot(h, W2_ref[0], preferred_element_type=jnp.float32)
         + b2_ref[0])
    out_ref[...] += w * o


@jax.jit
def kernel(x, gate_W, gate_b, W1, b1, W2, b2):
    grid = (N_TOKENS // BT, N_EXPERTS)
    return pl.pallas_call(
        _moe_dense_body,
        grid=grid,
        in_specs=[
            pl.BlockSpec((BT, D_MODEL), lambda t, e: (t, 0)),
            pl.BlockSpec((N_EXPERTS, D_MODEL), lambda t, e: (0, 0)),
            pl.BlockSpec((1, N_EXPERTS), lambda t, e: (0, 0)),
            pl.BlockSpec((1, D_MODEL, D_MODEL), lambda t, e: (e, 0, 0)),
            pl.BlockSpec((1, 1, D_MODEL), lambda t, e: (e, 0, 0)),
            pl.BlockSpec((1, D_MODEL, D_MODEL), lambda t, e: (e, 0, 0)),
            pl.BlockSpec((1, 1, D_MODEL), lambda t, e: (e, 0, 0)),
        ],
        out_specs=pl.BlockSpec((BT, D_MODEL), lambda t, e: (t, 0)),
        out_shape=jax.ShapeDtypeStruct((N_TOKENS, D_MODEL), jnp.float32),
        scratch_shapes=[pltpu.VMEM((BT, N_EXPERTS), jnp.float32)],
    )(x, gate_W, gate_b.reshape(1, N_EXPERTS),
      W1, b1.reshape(N_EXPERTS, 1, D_MODEL),
      W2, b2.reshape(N_EXPERTS, 1, D_MODEL))


# speedup vs baseline: 1.5750x; 1.3995x over previous
import functools

import jax
import jax.numpy as jnp
from jax.experimental import pallas as pl
from jax.experimental.pallas import tpu as pltpu

N_TOKENS = 2048
D_MODEL = 1024
N_EXPERTS = 8
BT = 256


def _moe_dense_body(x_ref, xb_ref, gw_ref, gb_ref, W1_ref, b1_ref, W2_ref,
                    b2_ref, out_ref):
    x = x_ref[...]
    idx = jax.lax.broadcasted_iota(jnp.int32, (BT, N_EXPERTS), 1)

    logits = jax.lax.dot_general(
        x, gw_ref[...], (((1,), (1,)), ((), ())),
        preferred_element_type=jnp.float32) + gb_ref[...]
    m1 = jnp.max(logits, axis=1, keepdims=True)
    i1 = jnp.min(jnp.where(logits == m1, idx, N_EXPERTS), axis=1,
                 keepdims=True)
    masked = jnp.where(idx == i1, -jnp.inf, logits)
    m2 = jnp.max(masked, axis=1, keepdims=True)
    i2 = jnp.min(jnp.where(masked == m2, idx, N_EXPERTS), axis=1,
                 keepdims=True)
    t = jnp.exp(m2 - m1)
    p1 = 1.0 / (1.0 + t)
    p2 = t / (1.0 + t)
    g = jnp.where(idx == i1, p1, jnp.where(idx == i2, p2, 0.0))

    xb = xb_ref[...]
    acc = jnp.zeros((BT, D_MODEL), jnp.float32)
    for e in range(N_EXPERTS):
        w = g[:, e:e + 1]
        h = jnp.maximum(
            jnp.dot(xb, W1_ref[e], preferred_element_type=jnp.float32)
            + b1_ref[e], 0.0)
        o = (jnp.dot(h.astype(jnp.bfloat16), W2_ref[e],
                     preferred_element_type=jnp.float32) + b2_ref[e])
        acc += w * o
    out_ref[...] = acc


@jax.jit
def kernel(x, gate_W, gate_b, W1, b1, W2, b2):
    grid = (N_TOKENS // BT,)
    return pl.pallas_call(
        _moe_dense_body,
        grid=grid,
        in_specs=[
            pl.BlockSpec((BT, D_MODEL), lambda t: (t, 0)),
            pl.BlockSpec((BT, D_MODEL), lambda t: (t, 0)),
            pl.BlockSpec((N_EXPERTS, D_MODEL), lambda t: (0, 0)),
            pl.BlockSpec((1, N_EXPERTS), lambda t: (0, 0)),
            pl.BlockSpec((N_EXPERTS, D_MODEL, D_MODEL), lambda t: (0, 0, 0)),
            pl.BlockSpec((N_EXPERTS, 1, D_MODEL), lambda t: (0, 0, 0)),
            pl.BlockSpec((N_EXPERTS, D_MODEL, D_MODEL), lambda t: (0, 0, 0)),
            pl.BlockSpec((N_EXPERTS, 1, D_MODEL), lambda t: (0, 0, 0)),
        ],
        out_specs=pl.BlockSpec((BT, D_MODEL), lambda t: (t, 0)),
        out_shape=jax.ShapeDtypeStruct((N_TOKENS, D_MODEL), jnp.float32),
    )(x, x.astype(jnp.bfloat16), gate_W, gate_b.reshape(1, N_EXPERTS),
      W1.astype(jnp.bfloat16), b1.reshape(N_EXPERTS, 1, D_MODEL),
      W2.astype(jnp.bfloat16), b2.reshape(N_EXPERTS, 1, D_MODEL))
